# baseline (speedup 1.0000x reference)
import functools

import jax
import jax.numpy as jnp
from jax import lax
from jax.experimental import pallas as pl
from jax.experimental.pallas import tpu as pltpu
from jax.experimental.pallas import tpu_sc as plsc

NU = 2000
NI = 8000
D = 256
E1 = 64000
EDGES = 2 * E1
NBT = NI * NU
EPT = EDGES // 16
NPASS = 10
CHUNK_I = NI // (2 * NPASS)
CHUNK_W = CHUNK_I * NU
WPT = CHUNK_W // 16
BOUNCE_W = WPT // 2


def _sc_mesh():
  return plsc.VectorSubcoreMesh(core_axis_name="c", subcore_axis_name="s")


def _build_b_body(eu_hbm, ei_hbm, b_hbm, g_v, idx_v, val_v,
                  zbuf, bounce, spm, sem, zsem):
  c = lax.axis_index("c")
  s = lax.axis_index("s")

  e0 = s * EPT
  pltpu.sync_copy(eu_hbm.at[pl.ds(e0, EPT)], g_v)
  pltpu.sync_copy(ei_hbm.at[pl.ds(e0, EPT)], idx_v)

  def _fold(k, carry):
    sl16 = pl.ds(k * 16, 16)
    g_v[sl16] = idx_v[sl16] * NU + g_v[sl16]
    return carry

  lax.fori_loop(0, EPT // 16, _fold, 0)

  z16 = jnp.zeros((16,), jnp.float32)

  def _zb(k, carry):
    zbuf[pl.ds(k * 16, 16)] = z16
    return carry

  lax.fori_loop(0, BOUNCE_W // 16, _zb, 0)

  iota16 = lax.iota(jnp.int32, 16)

  for q in range(NPASS):
    zcps = []
    for t in range(WPT // BOUNCE_W):
      cp = pltpu.make_async_copy(
          zbuf, spm.at[pl.ds(s * WPT + t * BOUNCE_W, BOUNCE_W)], zsem)
      cp.start()
      zcps.append(cp)

    gb0 = (c * NPASS + q) * CHUNK_W

    def _mk(k, carry):
      sl16 = pl.ds(k * 16, 16)
      rel = g_v[sl16] - gb0
      valid = (rel >= 0) & (rel < CHUNK_W)
      spread = k * 16 + iota16
      idx_v[sl16] = jnp.where(valid, rel, spread)
      val_v[sl16] = jnp.where(valid, 1.0, 0.0)
      return carry

    lax.fori_loop(0, EPT // 16, _mk, 0)

    for cp in zcps:
      cp.wait()
    plsc.subcore_barrier()
    pltpu.async_copy(val_v, spm.at[idx_v], sem, add=True).wait()
    plsc.subcore_barrier()

    qbase = (c * NPASS + q) * CHUNK_W
    for t in range(WPT // BOUNCE_W):
      off = s * WPT + t * BOUNCE_W
      pltpu.sync_copy(spm.at[pl.ds(off, BOUNCE_W)], bounce)
      pltpu.sync_copy(bounce, b_hbm.at[pl.ds(qbase + off, BOUNCE_W)])
    plsc.subcore_barrier()


def _build_b(edges_u, edges_i):
  k = pl.kernel(
      _build_b_body,
      out_type=jax.ShapeDtypeStruct((NBT,), jnp.float32),
      mesh=_sc_mesh(),
      scratch_types=[
          pltpu.VMEM((EPT,), jnp.int32),
          pltpu.VMEM((EPT,), jnp.int32),
          pltpu.VMEM((EPT,), jnp.float32),
          pltpu.VMEM((BOUNCE_W,), jnp.float32),
          pltpu.VMEM((BOUNCE_W,), jnp.float32),
          pltpu.VMEM_SHARED((CHUNK_W,), jnp.float32),
          pltpu.SemaphoreType.DMA,
          pltpu.SemaphoreType.DMA,
      ],
  )
  return k(edges_u, edges_i)


IT = 800
NA = NI // IT
XU_RT = NU // NA
XI_RT = NI // NA
ALPHA = 0.25


def _gcn_body(b_ref, xu_ref, xi_ref, ou_ref, oi_ref,
              dinvu, dinvi, zu, zi, nxu, nxi):
  p = pl.program_id(0)
  a = pl.program_id(1)
  bt = jnp.minimum(b_ref[...], 1.0)
  bt16 = bt.astype(jnp.bfloat16)
  sla = pl.ds(a * IT, IT)
  dn = (((0,), (0,)), ((), ()))

  @pl.when(p == 0)
  def _():
    dinvi[sla, :] = jnp.dot(bt, jnp.ones((NU, 1), jnp.float32),
                            preferred_element_type=jnp.float32)
    rsum = lax.dot_general(bt, jnp.ones((IT, 1), jnp.float32), dn,
                           preferred_element_type=jnp.float32)
    xu_b = xu_ref[...]
    xi_b = xi_ref[...]
    slu = pl.ds(a * XU_RT, XU_RT)
    sli = pl.ds(a * XI_RT, XI_RT)
    nxu[slu, :] = xu_b
    nxi[sli, :] = xi_b
    ou_ref[slu, :] = xu_b * ALPHA
    oi_ref[sli, :] = xi_b * ALPHA

    @pl.when(a == 0)
    def _():
      dinvu[...] = rsum

    @pl.when(a > 0)
    def _():
      dinvu[...] = dinvu[...] + rsum

  @pl.when((p == 1) & (a == 0))
  def _():
    dinvu[...] = jnp.where(dinvu[...] > 0, lax.rsqrt(dinvu[...]), 0.0)
    dinvi[...] = jnp.where(dinvi[...] > 0, lax.rsqrt(dinvi[...]), 0.0)
    zu[...] = (nxu[...] * dinvu[...]).astype(jnp.bfloat16)
    zi[...] = (nxi[...] * dinvi[...]).astype(jnp.bfloat16)

  @pl.when((p >= 2) & (a == 0))
  def _():
    t = dinvu[...] * nxu[...]
    ou_ref[...] = ou_ref[...] + ALPHA * t
    oi_ref[...] = oi_ref[...] + ALPHA * nxi[...]
    zu[...] = (dinvu[...] * t).astype(jnp.bfloat16)
    zi[...] = (dinvi[...] * nxi[...]).astype(jnp.bfloat16)

  @pl.when(p >= 1)
  def _():
    raw_i = jnp.dot(bt16, zu[...], preferred_element_type=jnp.float32)
    nxi[sla, :] = dinvi[sla, :] * raw_i
    contrib = lax.dot_general(bt16, zi[sla, :], dn,
                              preferred_element_type=jnp.float32)

    @pl.when(a == 0)
    def _():
      nxu[...] = contrib

    @pl.when(a > 0)
    def _():
      nxu[...] = nxu[...] + contrib

  @pl.when((p == 3) & (a == NA - 1))
  def _():
    ou_ref[...] = ou_ref[...] + ALPHA * (dinvu[...] * nxu[...])
    oi_ref[...] = oi_ref[...] + ALPHA * nxi[...]


def _gcn(bt2d, x_user, x_item):
  return pl.pallas_call(
      _gcn_body,
      grid=(4, NA),
      in_specs=[
          pl.BlockSpec((IT, NU), lambda p, a: (a, 0)),
          pl.BlockSpec((XU_RT, D),
                       lambda p, a: (jnp.where(p == 0, a, NA - 1), 0)),
          pl.BlockSpec((XI_RT, D),
                       lambda p, a: (jnp.where(p == 0, a, NA - 1), 0)),
      ],
      out_specs=[
          pl.BlockSpec((NU, D), lambda p, a: (0, 0)),
          pl.BlockSpec((NI, D), lambda p, a: (0, 0)),
      ],
      out_shape=[
          jax.ShapeDtypeStruct((NU, D), jnp.float32),
          jax.ShapeDtypeStruct((NI, D), jnp.float32),
      ],
      compiler_params=pltpu.CompilerParams(vmem_limit_bytes=63 << 20),
      scratch_shapes=[
          pltpu.VMEM((NU, 1), jnp.float32),
          pltpu.VMEM((NI, 1), jnp.float32),
          pltpu.VMEM((NU, D), jnp.bfloat16),
          pltpu.VMEM((NI, D), jnp.bfloat16),
          pltpu.VMEM((NU, D), jnp.float32),
          pltpu.VMEM((NI, D), jnp.float32),
      ],
  )(bt2d, x_user, x_item)


def kernel(x_user, x_item, edge_index_u2i, edge_index_i2u):
  edges_u = jnp.concatenate([edge_index_u2i[0], edge_index_i2u[1]])
  edges_i = jnp.concatenate([edge_index_u2i[1], edge_index_i2u[0]])
  b1d = _build_b(edges_u, edges_i)
  bt2d = b1d.reshape(NI, NU)
  out_u, out_i = _gcn(bt2d, x_user, x_item)
  return (out_u, out_i)


# speedup vs baseline: 1.0295x; 1.0295x over previous
import functools

import jax
import jax.numpy as jnp
from jax import lax
from jax.experimental import pallas as pl
from jax.experimental.pallas import tpu as pltpu
from jax.experimental.pallas import tpu_sc as plsc

NU = 2000
NI = 8000
D = 256
E1 = 64000
EDGES = 2 * E1
NBT = NI * NU
EPT = EDGES // 16
NPASS = 10
CHUNK_I = NI // (2 * NPASS)
CHUNK_E = CHUNK_I * NU
WPT = CHUNK_E // 16
BOUNCE_W = WPT // 2


def _sc_mesh():
  return plsc.VectorSubcoreMesh(core_axis_name="c", subcore_axis_name="s")


def _build_b_body(eu_hbm, ei_hbm, b_hbm, g_v, idx_v, ones_v,
                  zbuf, bounce, spm, sem, zsem):
  c = lax.axis_index("c")
  s = lax.axis_index("s")

  e0 = s * EPT
  pltpu.sync_copy(eu_hbm.at[pl.ds(e0, EPT)], g_v)
  pltpu.sync_copy(ei_hbm.at[pl.ds(e0, EPT)], idx_v)

  def _fold(k, carry):
    sl16 = pl.ds(k * 16, 16)
    g_v[sl16] = idx_v[sl16] * NU + g_v[sl16]
    return carry

  lax.fori_loop(0, EPT // 16, _fold, 0)

  one16 = jnp.ones((16,), jnp.float32)
  z16 = jnp.zeros((16,), jnp.float32)

  def _ob(k, carry):
    ones_v[pl.ds(k * 16, 16)] = one16
    return carry

  lax.fori_loop(0, EPT // 16, _ob, 0)

  def _zb(k, carry):
    zbuf[pl.ds(k * 16, 16)] = z16
    return carry

  lax.fori_loop(0, BOUNCE_W // 16, _zb, 0)

  iota16 = lax.iota(jnp.int32, 16)
  trash0 = CHUNK_E

  for q in range(NPASS):
    zcps = []
    for t in range(WPT // BOUNCE_W):
      cp = pltpu.make_async_copy(
          zbuf, spm.at[pl.ds(s * WPT + t * BOUNCE_W, BOUNCE_W)], zsem)
      cp.start()
      zcps.append(cp)

    gb0 = (c * NPASS + q) * CHUNK_E

    def _mk(k, carry):
      sl16 = pl.ds(k * 16, 16)
      rel = g_v[sl16] - gb0
      valid = (rel >= 0) & (rel < CHUNK_E)
      spread = trash0 + k * 16 + iota16
      idx_v[sl16] = jnp.where(valid, rel, spread)
      return carry

    lax.fori_loop(0, EPT // 16, _mk, 0)

    for cp in zcps:
      cp.wait()
    plsc.subcore_barrier()
    pltpu.async_copy(ones_v, spm.at[idx_v], sem, add=True).wait()
    plsc.subcore_barrier()

    qbase = (c * NPASS + q) * CHUNK_E
    for t in range(WPT // BOUNCE_W):
      off = s * WPT + t * BOUNCE_W
      pltpu.sync_copy(spm.at[pl.ds(off, BOUNCE_W)], bounce)
      pltpu.sync_copy(bounce, b_hbm.at[pl.ds(qbase + off, BOUNCE_W)])
    plsc.subcore_barrier()


def _build_b(edges_u, edges_i):
  k = pl.kernel(
      _build_b_body,
      out_type=jax.ShapeDtypeStruct((NBT,), jnp.float32),
      mesh=_sc_mesh(),
      scratch_types=[
          pltpu.VMEM((EPT,), jnp.int32),
          pltpu.VMEM((EPT,), jnp.int32),
          pltpu.VMEM((EPT,), jnp.float32),
          pltpu.VMEM((BOUNCE_W,), jnp.float32),
          pltpu.VMEM((BOUNCE_W,), jnp.float32),
          pltpu.VMEM_SHARED((CHUNK_E + EPT,), jnp.float32),
          pltpu.SemaphoreType.DMA,
          pltpu.SemaphoreType.DMA,
      ],
  )
  return k(edges_u, edges_i)


IT = 800
NA = NI // IT
XU_RT = NU // NA
XI_RT = NI // NA
ALPHA = 0.25


def _gcn_body(b_ref, xu_ref, xi_ref, ou_ref, oi_ref,
              dinvu, dinvi, zu, zi, nxu, nxi):
  p = pl.program_id(0)
  a = pl.program_id(1)
  bt = jnp.minimum(b_ref[...], 1.0)
  sla = pl.ds(a * IT, IT)
  dn = (((0,), (0,)), ((), ()))

  @pl.when(p == 0)
  def _():
    dinvi[sla, :] = jnp.dot(bt, jnp.ones((NU, 1), jnp.float32),
                            preferred_element_type=jnp.float32)
    rsum = lax.dot_general(bt, jnp.ones((IT, 1), jnp.float32), dn,
                           preferred_element_type=jnp.float32)
    xu_b = xu_ref[...]
    xi_b = xi_ref[...]
    slu = pl.ds(a * XU_RT, XU_RT)
    sli = pl.ds(a * XI_RT, XI_RT)
    nxu[slu, :] = xu_b
    nxi[sli, :] = xi_b
    ou_ref[slu, :] = xu_b * ALPHA
    oi_ref[sli, :] = xi_b * ALPHA

    @pl.when(a == 0)
    def _():
      dinvu[...] = rsum

    @pl.when(a > 0)
    def _():
      dinvu[...] = dinvu[...] + rsum

  @pl.when((p == 1) & (a == 0))
  def _():
    dinvu[...] = jnp.where(dinvu[...] > 0, lax.rsqrt(dinvu[...]), 0.0)
    dinvi[...] = jnp.where(dinvi[...] > 0, lax.rsqrt(dinvi[...]), 0.0)
    zu[...] = nxu[...] * dinvu[...]
    zi[...] = nxi[...] * dinvi[...]

  @pl.when((p >= 2) & (a == 0))
  def _():
    t = dinvu[...] * nxu[...]
    ou_ref[...] = ou_ref[...] + ALPHA * t
    oi_ref[...] = oi_ref[...] + ALPHA * nxi[...]
    zu[...] = dinvu[...] * t
    zi[...] = dinvi[...] * nxi[...]

  @pl.when(p >= 1)
  def _():
    raw_i = jnp.dot(bt, zu[...], preferred_element_type=jnp.float32)
    nxi[sla, :] = dinvi[sla, :] * raw_i
    contrib = lax.dot_general(bt, zi[sla, :], dn,
                              preferred_element_type=jnp.float32)

    @pl.when(a == 0)
    def _():
      nxu[...] = contrib

    @pl.when(a > 0)
    def _():
      nxu[...] = nxu[...] + contrib

  @pl.when((p == 3) & (a == NA - 1))
  def _():
    ou_ref[...] = ou_ref[...] + ALPHA * (dinvu[...] * nxu[...])
    oi_ref[...] = oi_ref[...] + ALPHA * nxi[...]


def _gcn(bt2d, x_user, x_item):
  return pl.pallas_call(
      _gcn_body,
      grid=(4, NA),
      in_specs=[
          pl.BlockSpec((IT, NU), lambda p, a: (a, 0)),
          pl.BlockSpec((XU_RT, D),
                       lambda p, a: (jnp.where(p == 0, a, NA - 1), 0)),
          pl.BlockSpec((XI_RT, D),
                       lambda p, a: (jnp.where(p == 0, a, NA - 1), 0)),
      ],
      out_specs=[
          pl.BlockSpec((NU, D), lambda p, a: (0, 0)),
          pl.BlockSpec((NI, D), lambda p, a: (0, 0)),
      ],
      out_shape=[
          jax.ShapeDtypeStruct((NU, D), jnp.float32),
          jax.ShapeDtypeStruct((NI, D), jnp.float32),
      ],
      compiler_params=pltpu.CompilerParams(vmem_limit_bytes=63 << 20),
      scratch_shapes=[
          pltpu.VMEM((NU, 1), jnp.float32),
          pltpu.VMEM((NI, 1), jnp.float32),
          pltpu.VMEM((NU, D), jnp.float32),
          pltpu.VMEM((NI, D), jnp.float32),
          pltpu.VMEM((NU, D), jnp.float32),
          pltpu.VMEM((NI, D), jnp.float32),
      ],
  )(bt2d, x_user, x_item)


def kernel(x_user, x_item, edge_index_u2i, edge_index_i2u):
  edges_u = jnp.concatenate([edge_index_u2i[0], edge_index_i2u[1]])
  edges_i = jnp.concatenate([edge_index_u2i[1], edge_index_i2u[0]])
  b1d = _build_b(edges_u, edges_i)
  bt2d = b1d.reshape(NI, NU)
  out_u, out_i = _gcn(bt2d, x_user, x_item)
  return (out_u, out_i)
